# Initial kernel scaffold; baseline (speedup 1.0000x reference)
import jax
import jax.numpy as jnp
from jax.experimental import pallas as pl


def kernel(embed_feat, h_gru, ei_follows, ei_likes, Win_f0, Win_f1, Wa_f0, Wa_f1, Win_l0, Win_l1, Wa_l0, Wa_l1, Wp1, bp1, Wp2):
    raise NotImplementedError("write your pallas kernel here")


# speedup vs baseline: 21.9301x; 21.9301x over previous
import functools
import jax
import jax.numpy as jnp
from jax import lax
from jax.experimental import pallas as pl
from jax.experimental.pallas import tpu as pltpu
from jax.experimental.pallas import tpu_sc as plsc

_N = 10000
_E = 160000
_D = 128
_OUT = 64
_NPAD = 10240
_NC = 2
_NS = 16
_CHUNK = 48
_CHUNKS_PER_TILE = 105
_EPT = _CHUNK * _CHUNKS_PER_TILE
_EPAD = _NC * _NS * _EPT
_NACC = 10112
_RPT = _NACC // _NS
_DEN = 20480
_DPT = _DEN // _NS


def _prep_body(x_ref, wf_ref, wl_ref, af_ref, al_ref,
               zf_ref, zl_ref, avf_ref, avl_ref):
    x = x_ref[...]
    zf = jnp.dot(x, wf_ref[...], preferred_element_type=jnp.float32)
    zl = jnp.dot(x, wl_ref[...], preferred_element_type=jnp.float32)
    zf_ref[...] = zf
    zl_ref[...] = zl
    avf_ref[...] = jnp.dot(zf, af_ref[...], preferred_element_type=jnp.float32)
    avl_ref[...] = jnp.dot(zl, al_ref[...], preferred_element_type=jnp.float32)


def _prep(x_pad, Wf, Wl, Af, Al):
    nblk = _NPAD // 1024
    return pl.pallas_call(
        _prep_body,
        grid=(nblk,),
        in_specs=[
            pl.BlockSpec((1024, _D), lambda i: (i, 0)),
            pl.BlockSpec((_D, _D), lambda i: (0, 0)),
            pl.BlockSpec((_D, _D), lambda i: (0, 0)),
            pl.BlockSpec((_D, 8), lambda i: (0, 0)),
            pl.BlockSpec((_D, 8), lambda i: (0, 0)),
        ],
        out_specs=[
            pl.BlockSpec((1024, _D), lambda i: (i, 0)),
            pl.BlockSpec((1024, _D), lambda i: (i, 0)),
            pl.BlockSpec((1024, 8), lambda i: (i, 0)),
            pl.BlockSpec((1024, 8), lambda i: (i, 0)),
        ],
        out_shape=[
            jax.ShapeDtypeStruct((_NPAD, _D), jnp.float32),
            jax.ShapeDtypeStruct((_NPAD, _D), jnp.float32),
            jax.ShapeDtypeStruct((_NPAD, 8), jnp.float32),
            jax.ShapeDtypeStruct((_NPAD, 8), jnp.float32),
        ],
    )(x_pad, Wf, Wl, Af, Al)


def _sc_body(zf, zl, af, al, pf, pl2, outf, outl, denf, denl,
             a_tab, pk_v, src_v, dst_v, d0i, d1i, ex0b, ex1b, rowbuf, zbuf,
             sem, acc_sh, den_sh):
    c = lax.axis_index("c")
    s = lax.axis_index("s")
    lane = lax.iota(jnp.int32, 16)
    zero16 = jnp.zeros((16,), jnp.float32)

    for k in range(8):
        zbuf[pl.ds(k * 16, 16)] = zero16

    for r in range(2):
        z_hbm = zf if r == 0 else zl
        a_hbm = af if r == 0 else al
        pk_hbm = pf if r == 0 else pl2
        out_hbm = outf if r == 0 else outl
        den_hbm = denf if r == 0 else denl

        pltpu.sync_copy(a_hbm, a_tab)

        def zrow(i, _):
            for k in range(_D // 16):
                rowbuf[i, pl.ds(k * 16, 16)] = zero16
            return 0
        lax.fori_loop(0, _CHUNK, zrow, 0)
        for t in range(_RPT // _CHUNK):
            pltpu.sync_copy(
                rowbuf, acc_sh.at[pl.ds(s * _RPT + t * _CHUNK, _CHUNK)])
        pltpu.sync_copy(rowbuf.at[pl.ds(0, _RPT % _CHUNK)],
                        acc_sh.at[pl.ds(s * _RPT + _RPT - _RPT % _CHUNK,
                                        _RPT % _CHUNK)])
        for t in range(_DPT // 128):
            pltpu.sync_copy(zbuf, den_sh.at[pl.ds(s * _DPT + t * 128, 128)])
        plsc.subcore_barrier()

        base0 = (c * _NS + s) * _EPT

        def chunk(j, _):
            base = base0 + j * _CHUNK
            pltpu.sync_copy(pk_hbm.at[pl.ds(base, _CHUNK)], pk_v)
            for g in range(_CHUNK // 16):
                p16 = pk_v[pl.ds(g * 16, 16)]
                s16 = lax.bitwise_and(p16, 16383)
                d16 = lax.shift_right_logical(p16, 14)
                src_v[pl.ds(g * 16, 16)] = s16
                dst_v[pl.ds(g * 16, 16)] = d16
                as0 = plsc.load_gather(a_tab, [s16 * 4])
                ad0 = plsc.load_gather(a_tab, [d16 * 4 + 1])
                as1 = plsc.load_gather(a_tab, [s16 * 4 + 2])
                ad1 = plsc.load_gather(a_tab, [d16 * 4 + 3])
                e0 = as0 + ad0
                e1 = as1 + ad1
                e0 = jnp.where(e0 >= 0.0, e0, 0.01 * e0)
                e1 = jnp.where(e1 >= 0.0, e1, 0.01 * e1)
                live = (base + g * 16 + lane) < _E
                ex0b[pl.ds(g * 16, 16)] = jnp.where(live, jnp.exp(e0), 0.0)
                ex1b[pl.ds(g * 16, 16)] = jnp.where(live, jnp.exp(e1), 0.0)
                d0i[pl.ds(g * 16, 16)] = d16 * 2
                d1i[pl.ds(g * 16, 16)] = d16 * 2 + 1
            pltpu.async_copy(z_hbm.at[src_v], rowbuf, sem).wait()
            for g2 in range(_CHUNK // 16):
                exv0 = ex0b[pl.ds(g2 * 16, 16)]
                exv1 = ex1b[pl.ds(g2 * 16, 16)]
                for l in range(16):
                    i = g2 * 16 + l
                    b0 = jnp.broadcast_to(exv0[l], (16,))
                    b1 = jnp.broadcast_to(exv1[l], (16,))
                    for k in range(4):
                        rowbuf[i, pl.ds(k * 16, 16)] = (
                            rowbuf[i, pl.ds(k * 16, 16)] * b0)
                    for k in range(4, 8):
                        rowbuf[i, pl.ds(k * 16, 16)] = (
                            rowbuf[i, pl.ds(k * 16, 16)] * b1)
            pltpu.sync_copy(rowbuf, acc_sh.at[dst_v], add=True)
            pltpu.sync_copy(ex0b, den_sh.at[d0i], add=True)
            pltpu.sync_copy(ex1b, den_sh.at[d1i], add=True)
            return 0

        lax.fori_loop(0, _CHUNKS_PER_TILE, chunk, 0)
        plsc.subcore_barrier()
        pltpu.sync_copy(acc_sh.at[pl.ds(s * _RPT, _RPT)],
                        out_hbm.at[c, pl.ds(s * _RPT, _RPT)])
        pltpu.sync_copy(den_sh.at[pl.ds(s * _DPT, _DPT)],
                        den_hbm.at[c, pl.ds(s * _DPT, _DPT)])
        plsc.subcore_barrier()


def _sc_aggregate(z_f, z_l, a_f, a_l, pk_f, pk_l):
    mesh = plsc.VectorSubcoreMesh(core_axis_name="c", subcore_axis_name="s")
    kern = functools.partial(
        pl.kernel,
        mesh=mesh,
        compiler_params=pltpu.CompilerParams(needs_layout_passes=False),
        out_type=[
            jax.ShapeDtypeStruct((_NC, _NACC, _D), jnp.float32),
            jax.ShapeDtypeStruct((_NC, _NACC, _D), jnp.float32),
            jax.ShapeDtypeStruct((_NC, _DEN), jnp.float32),
            jax.ShapeDtypeStruct((_NC, _DEN), jnp.float32),
        ],
        scratch_types=[
            pltpu.VMEM((_N * 4,), jnp.float32),
            pltpu.VMEM((_CHUNK,), jnp.int32),
            pltpu.VMEM((_CHUNK,), jnp.int32),
            pltpu.VMEM((_CHUNK,), jnp.int32),
            pltpu.VMEM((_CHUNK,), jnp.int32),
            pltpu.VMEM((_CHUNK,), jnp.int32),
            pltpu.VMEM((_CHUNK,), jnp.float32),
            pltpu.VMEM((_CHUNK,), jnp.float32),
            pltpu.VMEM((_CHUNK, _D), jnp.float32),
            pltpu.VMEM((128,), jnp.float32),
            pltpu.SemaphoreType.DMA,
            pltpu.VMEM_SHARED((_NACC, _D), jnp.float32),
            pltpu.VMEM_SHARED((_DEN,), jnp.float32),
        ],
    )(_sc_body)
    return kern(z_f, z_l, a_f, a_l, pk_f, pk_l)


def _combine_body(nf_ref, nl_ref, df_ref, dl_ref, wp1_ref, bp1_ref,
                  wp2_ref, attf_ref, attl_ref, ws_ref):
    i = pl.program_id(0)

    def att_of(a_ref, d_ref):
        n = a_ref[0] + a_ref[1]
        d = d_ref[0] + d_ref[1]
        d0 = d[:, 0:1]
        d1 = d[:, 1:2]
        d0 = jnp.where(d0 > 0.0, d0, 1.0)
        d1 = jnp.where(d1 > 0.0, d1, 1.0)
        return jnp.concatenate([n[:, :64] / d0, n[:, 64:128] / d1], axis=1)

    attf = att_of(nf_ref, df_ref)
    attl = att_of(nl_ref, dl_ref)
    attf_ref[...] = attf
    attl_ref[...] = attl

    wp1 = wp1_ref[...]
    bp1 = bp1_ref[...]
    wp2 = wp2_ref[...]

    def wsum(att):
        t = jnp.tanh(jnp.dot(att, wp1, preferred_element_type=jnp.float32)
                     + bp1)
        return jnp.sum(t * wp2)

    @pl.when(i == 0)
    def _():
        ws_ref[0, 0] = 0.0
        ws_ref[0, 1] = 0.0

    ws_ref[0, 0] = ws_ref[0, 0] + wsum(attf)
    ws_ref[0, 1] = ws_ref[0, 1] + wsum(attl)


def _combine(acc_f, acc_l, den_f, den_l, Wp1T, bp1r, Wp2r):
    blk = 1000
    nblk = _N // blk
    return pl.pallas_call(
        _combine_body,
        grid=(nblk,),
        in_specs=[
            pl.BlockSpec((2, blk, _D), lambda i: (0, i, 0)),
            pl.BlockSpec((2, blk, _D), lambda i: (0, i, 0)),
            pl.BlockSpec((2, blk, 2), lambda i: (0, i, 0)),
            pl.BlockSpec((2, blk, 2), lambda i: (0, i, 0)),
            pl.BlockSpec((_D, _D), lambda i: (0, 0)),
            pl.BlockSpec((1, _D), lambda i: (0, 0)),
            pl.BlockSpec((1, _D), lambda i: (0, 0)),
        ],
        out_specs=[
            pl.BlockSpec((blk, _D), lambda i: (i, 0)),
            pl.BlockSpec((blk, _D), lambda i: (i, 0)),
            pl.BlockSpec((1, 2), lambda i: (0, 0), memory_space=pltpu.SMEM),
        ],
        out_shape=[
            jax.ShapeDtypeStruct((_N, _D), jnp.float32),
            jax.ShapeDtypeStruct((_N, _D), jnp.float32),
            jax.ShapeDtypeStruct((1, 2), jnp.float32),
        ],
    )(acc_f, acc_l, den_f, den_l, Wp1T, bp1r, Wp2r)


def _final_body(attf_ref, attl_ref, beta_ref, out_ref):
    bf = beta_ref[0, 0]
    bl = beta_ref[0, 1]
    out_ref[...] = bf * attf_ref[...] + bl * attl_ref[...]


def _final(attf, attl, beta):
    blk = 1000
    nblk = _N // blk
    return pl.pallas_call(
        _final_body,
        grid=(nblk,),
        in_specs=[
            pl.BlockSpec((blk, _D), lambda i: (i, 0)),
            pl.BlockSpec((blk, _D), lambda i: (i, 0)),
            pl.BlockSpec((1, 2), lambda i: (0, 0), memory_space=pltpu.SMEM),
        ],
        out_specs=pl.BlockSpec((blk, _D), lambda i: (i, 0)),
        out_shape=jax.ShapeDtypeStruct((_N, _D), jnp.float32),
    )(attf, attl, beta)


def _pad_edges(ei):
    pad_n = _EPAD - _E
    ar = jnp.arange(pad_n, dtype=jnp.int32)
    pad_src = ar % _N
    pad_dst = (ar * 37) % _N
    src = jnp.concatenate([ei[0].astype(jnp.int32), pad_src])
    dst = jnp.concatenate([ei[1].astype(jnp.int32), pad_dst])
    return dst * 16384 + src


@jax.jit
def kernel(embed_feat, h_gru, ei_follows, ei_likes, Win_f0, Win_f1, Wa_f0,
           Wa_f1, Win_l0, Win_l1, Wa_l0, Wa_l1, Wp1, bp1, Wp2):
    x = jnp.concatenate([embed_feat, h_gru], axis=-1)
    x_pad = jnp.pad(x, ((0, _NPAD - _N), (0, 0)))

    Wf = jnp.concatenate([Win_f0, Win_f1], axis=0).T
    Wl = jnp.concatenate([Win_l0, Win_l1], axis=0).T
    zvec = jnp.zeros((_OUT,), jnp.float32)

    def amat(Wa0, Wa1):
        c0 = jnp.concatenate([Wa0[0, :_OUT], zvec])
        c1 = jnp.concatenate([Wa0[0, _OUT:], zvec])
        c2 = jnp.concatenate([zvec, Wa1[0, :_OUT]])
        c3 = jnp.concatenate([zvec, Wa1[0, _OUT:]])
        z4 = jnp.zeros((_D, 4), jnp.float32)
        return jnp.concatenate(
            [jnp.stack([c0, c1, c2, c3], axis=1), z4], axis=1)

    Af = amat(Wa_f0, Wa_f1)
    Al = amat(Wa_l0, Wa_l1)

    z_f, z_l, av_f, av_l = _prep(x_pad, Wf, Wl, Af, Al)
    a_f = av_f[:_N, :4].reshape(-1)
    a_l = av_l[:_N, :4].reshape(-1)

    pk_f = _pad_edges(ei_follows)
    pk_l = _pad_edges(ei_likes)

    acc_f, acc_l, den_f, den_l = _sc_aggregate(z_f, z_l, a_f, a_l,
                                               pk_f, pk_l)
    den_f = den_f[:, :2 * _N].reshape(_NC, _N, 2)
    den_l = den_l[:, :2 * _N].reshape(_NC, _N, 2)

    Wp1T = Wp1.T
    bp1r = bp1.reshape(1, _D)
    Wp2r = Wp2.reshape(1, _D)
    attf, attl, ws = _combine(acc_f, acc_l, den_f, den_l, Wp1T, bp1r, Wp2r)

    wf = ws[0, 0] / _N
    wl = ws[0, 1] / _N
    m = jnp.maximum(wf, wl)
    ef = jnp.exp(wf - m)
    el = jnp.exp(wl - m)
    beta = jnp.stack([ef, el]) / (ef + el)
    return _final(attf, attl, beta.reshape(1, 2))


# speedup vs baseline: 27.2180x; 1.2411x over previous
import functools
import jax
import jax.numpy as jnp
from jax import lax
from jax.experimental import pallas as pl
from jax.experimental.pallas import tpu as pltpu
from jax.experimental.pallas import tpu_sc as plsc

_N = 10000
_E = 160000
_D = 128
_OUT = 64
_NPAD = 10240
_NC = 2
_NS = 16
_CHUNK = 48
_CHUNKS_PER_TILE = 105
_EPT = _CHUNK * _CHUNKS_PER_TILE
_EPAD = _NC * _NS * _EPT
_NACC = 10112
_RPT = _NACC // _NS
_DEN = 20480
_DPT = _DEN // _NS


def _prep_body(x_ref, wf_ref, wl_ref, af_ref, al_ref,
               zf_ref, zl_ref, avf_ref, avl_ref):
    x = x_ref[...]
    zf = jnp.dot(x, wf_ref[...], preferred_element_type=jnp.float32)
    zl = jnp.dot(x, wl_ref[...], preferred_element_type=jnp.float32)
    zf_ref[...] = zf
    zl_ref[...] = zl
    avf_ref[...] = jnp.dot(zf, af_ref[...], preferred_element_type=jnp.float32)
    avl_ref[...] = jnp.dot(zl, al_ref[...], preferred_element_type=jnp.float32)


def _prep(x_pad, Wf, Wl, Af, Al):
    nblk = _NPAD // 1024
    return pl.pallas_call(
        _prep_body,
        grid=(nblk,),
        in_specs=[
            pl.BlockSpec((1024, _D), lambda i: (i, 0)),
            pl.BlockSpec((_D, _D), lambda i: (0, 0)),
            pl.BlockSpec((_D, _D), lambda i: (0, 0)),
            pl.BlockSpec((_D, 8), lambda i: (0, 0)),
            pl.BlockSpec((_D, 8), lambda i: (0, 0)),
        ],
        out_specs=[
            pl.BlockSpec((1024, _D), lambda i: (i, 0)),
            pl.BlockSpec((1024, _D), lambda i: (i, 0)),
            pl.BlockSpec((1024, 8), lambda i: (i, 0)),
            pl.BlockSpec((1024, 8), lambda i: (i, 0)),
        ],
        out_shape=[
            jax.ShapeDtypeStruct((_NPAD, _D), jnp.float32),
            jax.ShapeDtypeStruct((_NPAD, _D), jnp.float32),
            jax.ShapeDtypeStruct((_NPAD, 8), jnp.float32),
            jax.ShapeDtypeStruct((_NPAD, 8), jnp.float32),
        ],
    )(x_pad, Wf, Wl, Af, Al)


def _sc_body(zf, zl, af, al, pf, pl2, outf, outl, denf, denl,
             a_tab, pk_v, src_v, dst_v, d0i, d1i, ex0b, ex1b, rowbuf, zbuf,
             sem, pk_sem, sc_sem, d_sem, acc_sh, den_sh):
    c = lax.axis_index("c")
    s = lax.axis_index("s")
    lane = lax.iota(jnp.int32, 16)
    zero16 = jnp.zeros((16,), jnp.float32)

    for k in range(8):
        zbuf[pl.ds(k * 16, 16)] = zero16

    for r in range(2):
        z_hbm = zf if r == 0 else zl
        a_hbm = af if r == 0 else al
        pk_hbm = pf if r == 0 else pl2
        out_hbm = outf if r == 0 else outl
        den_hbm = denf if r == 0 else denl

        pltpu.sync_copy(a_hbm, a_tab)

        def zrow(i, _):
            for k in range(_D // 16):
                rowbuf[i, pl.ds(k * 16, 16)] = zero16
            return 0
        lax.fori_loop(0, _CHUNK, zrow, 0)
        for t in range(_RPT // _CHUNK):
            pltpu.sync_copy(
                rowbuf, acc_sh.at[pl.ds(s * _RPT + t * _CHUNK, _CHUNK)])
        pltpu.sync_copy(rowbuf.at[pl.ds(0, _RPT % _CHUNK)],
                        acc_sh.at[pl.ds(s * _RPT + _RPT - _RPT % _CHUNK,
                                        _RPT % _CHUNK)])
        for t in range(_DPT // 128):
            pltpu.sync_copy(zbuf, den_sh.at[pl.ds(s * _DPT + t * 128, 128)])
        plsc.subcore_barrier()

        base0 = (c * _NS + s) * _EPT
        pltpu.async_copy(pk_hbm.at[pl.ds(base0, _CHUNK)], pk_v, pk_sem)

        def chunk(j, _):
            base = base0 + j * _CHUNK
            pltpu.make_async_copy(
                pk_hbm.at[pl.ds(base, _CHUNK)], pk_v, pk_sem).wait()

            @pl.when(j > 0)
            def _():
                pltpu.make_async_copy(
                    rowbuf, acc_sh.at[dst_v], sc_sem).wait()
                pltpu.make_async_copy(ex0b, den_sh.at[d0i], d_sem).wait()
                pltpu.make_async_copy(ex1b, den_sh.at[d1i], d_sem).wait()

            for g in range(_CHUNK // 16):
                p16 = pk_v[pl.ds(g * 16, 16)]
                s16 = lax.bitwise_and(p16, 16383)
                d16 = lax.shift_right_logical(p16, 14)
                src_v[pl.ds(g * 16, 16)] = s16
                dst_v[pl.ds(g * 16, 16)] = d16
                as0 = plsc.load_gather(a_tab, [s16 * 4])
                ad0 = plsc.load_gather(a_tab, [d16 * 4 + 1])
                as1 = plsc.load_gather(a_tab, [s16 * 4 + 2])
                ad1 = plsc.load_gather(a_tab, [d16 * 4 + 3])
                e0 = as0 + ad0
                e1 = as1 + ad1
                e0 = jnp.where(e0 >= 0.0, e0, 0.01 * e0)
                e1 = jnp.where(e1 >= 0.0, e1, 0.01 * e1)
                live = (base + g * 16 + lane) < _E
                ex0b[pl.ds(g * 16, 16)] = jnp.where(live, jnp.exp(e0), 0.0)
                ex1b[pl.ds(g * 16, 16)] = jnp.where(live, jnp.exp(e1), 0.0)
                d0i[pl.ds(g * 16, 16)] = d16 * 2
                d1i[pl.ds(g * 16, 16)] = d16 * 2 + 1
            @pl.when(j < _CHUNKS_PER_TILE - 1)
            def _():
                pltpu.async_copy(
                    pk_hbm.at[pl.ds(base + _CHUNK, _CHUNK)], pk_v, pk_sem)
            pltpu.async_copy(ex0b, den_sh.at[d0i], d_sem, add=True)
            pltpu.async_copy(ex1b, den_sh.at[d1i], d_sem, add=True)
            pltpu.async_copy(z_hbm.at[src_v], rowbuf, sem).wait()
            for g2 in range(_CHUNK // 16):
                exv0 = ex0b[pl.ds(g2 * 16, 16)]
                exv1 = ex1b[pl.ds(g2 * 16, 16)]
                for l in range(16):
                    i = g2 * 16 + l
                    b0 = jnp.broadcast_to(exv0[l], (16,))
                    b1 = jnp.broadcast_to(exv1[l], (16,))
                    for k in range(4):
                        rowbuf[i, pl.ds(k * 16, 16)] = (
                            rowbuf[i, pl.ds(k * 16, 16)] * b0)
                    for k in range(4, 8):
                        rowbuf[i, pl.ds(k * 16, 16)] = (
                            rowbuf[i, pl.ds(k * 16, 16)] * b1)
            pltpu.async_copy(rowbuf, acc_sh.at[dst_v], sc_sem, add=True)
            return 0

        lax.fori_loop(0, _CHUNKS_PER_TILE, chunk, 0)
        pltpu.make_async_copy(rowbuf, acc_sh.at[dst_v], sc_sem).wait()
        pltpu.make_async_copy(ex0b, den_sh.at[d0i], d_sem).wait()
        pltpu.make_async_copy(ex1b, den_sh.at[d1i], d_sem).wait()
        plsc.subcore_barrier()
        pltpu.sync_copy(acc_sh.at[pl.ds(s * _RPT, _RPT)],
                        out_hbm.at[c, pl.ds(s * _RPT, _RPT)])
        pltpu.sync_copy(den_sh.at[pl.ds(s * _DPT, _DPT)],
                        den_hbm.at[c, pl.ds(s * _DPT, _DPT)])
        plsc.subcore_barrier()


def _sc_aggregate(z_f, z_l, a_f, a_l, pk_f, pk_l):
    mesh = plsc.VectorSubcoreMesh(core_axis_name="c", subcore_axis_name="s")
    kern = functools.partial(
        pl.kernel,
        mesh=mesh,
        compiler_params=pltpu.CompilerParams(needs_layout_passes=False),
        out_type=[
            jax.ShapeDtypeStruct((_NC, _NACC, _D), jnp.float32),
            jax.ShapeDtypeStruct((_NC, _NACC, _D), jnp.float32),
            jax.ShapeDtypeStruct((_NC, _DEN), jnp.float32),
            jax.ShapeDtypeStruct((_NC, _DEN), jnp.float32),
        ],
        scratch_types=[
            pltpu.VMEM((_N * 4,), jnp.float32),
            pltpu.VMEM((_CHUNK,), jnp.int32),
            pltpu.VMEM((_CHUNK,), jnp.int32),
            pltpu.VMEM((_CHUNK,), jnp.int32),
            pltpu.VMEM((_CHUNK,), jnp.int32),
            pltpu.VMEM((_CHUNK,), jnp.int32),
            pltpu.VMEM((_CHUNK,), jnp.float32),
            pltpu.VMEM((_CHUNK,), jnp.float32),
            pltpu.VMEM((_CHUNK, _D), jnp.float32),
            pltpu.VMEM((128,), jnp.float32),
            pltpu.SemaphoreType.DMA,
            pltpu.SemaphoreType.DMA,
            pltpu.SemaphoreType.DMA,
            pltpu.SemaphoreType.DMA,
            pltpu.VMEM_SHARED((_NACC, _D), jnp.float32),
            pltpu.VMEM_SHARED((_DEN,), jnp.float32),
        ],
    )(_sc_body)
    return kern(z_f, z_l, a_f, a_l, pk_f, pk_l)


def _combine_body(nf_ref, nl_ref, df_ref, dl_ref, wp1_ref, bp1_ref,
                  wp2_ref, attf_ref, attl_ref, ws_ref):
    i = pl.program_id(0)

    def att_of(a_ref, d_ref):
        n = a_ref[0] + a_ref[1]
        d = d_ref[0] + d_ref[1]
        d0 = d[:, 0:1]
        d1 = d[:, 1:2]
        d0 = jnp.where(d0 > 0.0, d0, 1.0)
        d1 = jnp.where(d1 > 0.0, d1, 1.0)
        return jnp.concatenate([n[:, :64] / d0, n[:, 64:128] / d1], axis=1)

    attf = att_of(nf_ref, df_ref)
    attl = att_of(nl_ref, dl_ref)
    attf_ref[...] = attf
    attl_ref[...] = attl

    wp1 = wp1_ref[...]
    bp1 = bp1_ref[...]
    wp2 = wp2_ref[...]

    def wsum(att):
        t = jnp.tanh(jnp.dot(att, wp1, preferred_element_type=jnp.float32)
                     + bp1)
        return jnp.sum(t * wp2)

    @pl.when(i == 0)
    def _():
        ws_ref[0, 0] = 0.0
        ws_ref[0, 1] = 0.0

    ws_ref[0, 0] = ws_ref[0, 0] + wsum(attf)
    ws_ref[0, 1] = ws_ref[0, 1] + wsum(attl)


def _combine(acc_f, acc_l, den_f, den_l, Wp1T, bp1r, Wp2r):
    blk = 1000
    nblk = _N // blk
    return pl.pallas_call(
        _combine_body,
        grid=(nblk,),
        in_specs=[
            pl.BlockSpec((2, blk, _D), lambda i: (0, i, 0)),
            pl.BlockSpec((2, blk, _D), lambda i: (0, i, 0)),
            pl.BlockSpec((2, blk, 2), lambda i: (0, i, 0)),
            pl.BlockSpec((2, blk, 2), lambda i: (0, i, 0)),
            pl.BlockSpec((_D, _D), lambda i: (0, 0)),
            pl.BlockSpec((1, _D), lambda i: (0, 0)),
            pl.BlockSpec((1, _D), lambda i: (0, 0)),
        ],
        out_specs=[
            pl.BlockSpec((blk, _D), lambda i: (i, 0)),
            pl.BlockSpec((blk, _D), lambda i: (i, 0)),
            pl.BlockSpec((1, 2), lambda i: (0, 0), memory_space=pltpu.SMEM),
        ],
        out_shape=[
            jax.ShapeDtypeStruct((_N, _D), jnp.float32),
            jax.ShapeDtypeStruct((_N, _D), jnp.float32),
            jax.ShapeDtypeStruct((1, 2), jnp.float32),
        ],
    )(acc_f, acc_l, den_f, den_l, Wp1T, bp1r, Wp2r)


def _final_body(attf_ref, attl_ref, beta_ref, out_ref):
    bf = beta_ref[0, 0]
    bl = beta_ref[0, 1]
    out_ref[...] = bf * attf_ref[...] + bl * attl_ref[...]


def _final(attf, attl, beta):
    blk = 1000
    nblk = _N // blk
    return pl.pallas_call(
        _final_body,
        grid=(nblk,),
        in_specs=[
            pl.BlockSpec((blk, _D), lambda i: (i, 0)),
            pl.BlockSpec((blk, _D), lambda i: (i, 0)),
            pl.BlockSpec((1, 2), lambda i: (0, 0), memory_space=pltpu.SMEM),
        ],
        out_specs=pl.BlockSpec((blk, _D), lambda i: (i, 0)),
        out_shape=jax.ShapeDtypeStruct((_N, _D), jnp.float32),
    )(attf, attl, beta)


def _pad_edges(ei):
    pad_n = _EPAD - _E
    ar = jnp.arange(pad_n, dtype=jnp.int32)
    pad_src = ar % _N
    pad_dst = (ar * 37) % _N
    src = jnp.concatenate([ei[0].astype(jnp.int32), pad_src])
    dst = jnp.concatenate([ei[1].astype(jnp.int32), pad_dst])
    return dst * 16384 + src


@jax.jit
def kernel(embed_feat, h_gru, ei_follows, ei_likes, Win_f0, Win_f1, Wa_f0,
           Wa_f1, Win_l0, Win_l1, Wa_l0, Wa_l1, Wp1, bp1, Wp2):
    x = jnp.concatenate([embed_feat, h_gru], axis=-1)
    x_pad = jnp.pad(x, ((0, _NPAD - _N), (0, 0)))

    Wf = jnp.concatenate([Win_f0, Win_f1], axis=0).T
    Wl = jnp.concatenate([Win_l0, Win_l1], axis=0).T
    zvec = jnp.zeros((_OUT,), jnp.float32)

    def amat(Wa0, Wa1):
        c0 = jnp.concatenate([Wa0[0, :_OUT], zvec])
        c1 = jnp.concatenate([Wa0[0, _OUT:], zvec])
        c2 = jnp.concatenate([zvec, Wa1[0, :_OUT]])
        c3 = jnp.concatenate([zvec, Wa1[0, _OUT:]])
        z4 = jnp.zeros((_D, 4), jnp.float32)
        return jnp.concatenate(
            [jnp.stack([c0, c1, c2, c3], axis=1), z4], axis=1)

    Af = amat(Wa_f0, Wa_f1)
    Al = amat(Wa_l0, Wa_l1)

    z_f, z_l, av_f, av_l = _prep(x_pad, Wf, Wl, Af, Al)
    a_f = av_f[:_N, :4].reshape(-1)
    a_l = av_l[:_N, :4].reshape(-1)

    pk_f = _pad_edges(ei_follows)
    pk_l = _pad_edges(ei_likes)

    acc_f, acc_l, den_f, den_l = _sc_aggregate(z_f, z_l, a_f, a_l,
                                               pk_f, pk_l)
    den_f = den_f[:, :2 * _N].reshape(_NC, _N, 2)
    den_l = den_l[:, :2 * _N].reshape(_NC, _N, 2)

    Wp1T = Wp1.T
    bp1r = bp1.reshape(1, _D)
    Wp2r = Wp2.reshape(1, _D)
    attf, attl, ws = _combine(acc_f, acc_l, den_f, den_l, Wp1T, bp1r, Wp2r)

    wf = ws[0, 0] / _N
    wl = ws[0, 1] / _N
    m = jnp.maximum(wf, wl)
    ef = jnp.exp(wf - m)
    el = jnp.exp(wl - m)
    beta = jnp.stack([ef, el]) / (ef + el)
    return _final(attf, attl, beta.reshape(1, 2))


# speedup vs baseline: 29.4063x; 1.0804x over previous
import functools
import jax
import jax.numpy as jnp
from jax import lax
from jax.experimental import pallas as pl
from jax.experimental.pallas import tpu as pltpu
from jax.experimental.pallas import tpu_sc as plsc

_N = 10000
_E = 160000
_D = 128
_OUT = 64
_NPAD = 10240
_NC = 2
_NS = 16
_CHUNK = 48
_CHUNKS_PER_TILE = 105
_EPT = _CHUNK * _CHUNKS_PER_TILE
_EPAD = _NC * _NS * _EPT
_NACC = 10112
_RPT = _NACC // _NS
_DEN = 20480
_DPT = _DEN // _NS


def _prep_body(x_ref, wf_ref, wl_ref, af_ref, al_ref,
               zf_ref, zl_ref, avf_ref, avl_ref):
    x = x_ref[...]
    zf = jnp.dot(x, wf_ref[...], preferred_element_type=jnp.float32)
    zl = jnp.dot(x, wl_ref[...], preferred_element_type=jnp.float32)
    zf_ref[...] = zf
    zl_ref[...] = zl
    avf_ref[...] = jnp.dot(zf, af_ref[...], preferred_element_type=jnp.float32)
    avl_ref[...] = jnp.dot(zl, al_ref[...], preferred_element_type=jnp.float32)


def _prep(x_pad, Wf, Wl, Af, Al):
    nblk = _NPAD // 1024
    return pl.pallas_call(
        _prep_body,
        grid=(nblk,),
        in_specs=[
            pl.BlockSpec((1024, _D), lambda i: (i, 0)),
            pl.BlockSpec((_D, _D), lambda i: (0, 0)),
            pl.BlockSpec((_D, _D), lambda i: (0, 0)),
            pl.BlockSpec((_D, 8), lambda i: (0, 0)),
            pl.BlockSpec((_D, 8), lambda i: (0, 0)),
        ],
        out_specs=[
            pl.BlockSpec((1024, _D), lambda i: (i, 0)),
            pl.BlockSpec((1024, _D), lambda i: (i, 0)),
            pl.BlockSpec((1024, 8), lambda i: (i, 0)),
            pl.BlockSpec((1024, 8), lambda i: (i, 0)),
        ],
        out_shape=[
            jax.ShapeDtypeStruct((_NPAD, _D), jnp.float32),
            jax.ShapeDtypeStruct((_NPAD, _D), jnp.float32),
            jax.ShapeDtypeStruct((_NPAD, 8), jnp.float32),
            jax.ShapeDtypeStruct((_NPAD, 8), jnp.float32),
        ],
    )(x_pad, Wf, Wl, Af, Al)


def _sc_body(zf, zl, af, al, pf, pl2, outf, outl, denf, denl,
             a_tab, pk_v, sg0, sg1, sg2, dst_v, d0i, d1i, ex0b, ex1b,
             rowbuf, zbuf, sem, pk_sem, sc_sem, d_sem, acc_sh, den_sh):
    c = lax.axis_index("c")
    s = lax.axis_index("s")
    lane = lax.iota(jnp.int32, 16)
    zero16 = jnp.zeros((16,), jnp.float32)

    for k in range(8):
        zbuf[pl.ds(k * 16, 16)] = zero16

    for r in range(2):
        z_hbm = zf if r == 0 else zl
        a_hbm = af if r == 0 else al
        pk_hbm = pf if r == 0 else pl2
        out_hbm = outf if r == 0 else outl
        den_hbm = denf if r == 0 else denl

        pltpu.sync_copy(a_hbm, a_tab)

        def zrow(i, _):
            for k in range(_D // 16):
                rowbuf[i, pl.ds(k * 16, 16)] = zero16
            return 0
        lax.fori_loop(0, _CHUNK, zrow, 0)
        for t in range(_RPT // _CHUNK):
            pltpu.sync_copy(
                rowbuf, acc_sh.at[pl.ds(s * _RPT + t * _CHUNK, _CHUNK)])
        pltpu.sync_copy(rowbuf.at[pl.ds(0, _RPT % _CHUNK)],
                        acc_sh.at[pl.ds(s * _RPT + _RPT - _RPT % _CHUNK,
                                        _RPT % _CHUNK)])
        for t in range(_DPT // 128):
            pltpu.sync_copy(zbuf, den_sh.at[pl.ds(s * _DPT + t * 128, 128)])
        plsc.subcore_barrier()

        base0 = (c * _NS + s) * _EPT
        pltpu.async_copy(pk_hbm.at[pl.ds(base0, _CHUNK)], pk_v, pk_sem)

        def chunk(j, _):
            base = base0 + j * _CHUNK
            pltpu.make_async_copy(
                pk_hbm.at[pl.ds(base, _CHUNK)], pk_v, pk_sem).wait()

            @pl.when(j > 0)
            def _():
                pltpu.make_async_copy(
                    rowbuf, acc_sh.at[dst_v], sc_sem).wait()
                pltpu.make_async_copy(ex0b, den_sh.at[d0i], d_sem).wait()
                pltpu.make_async_copy(ex1b, den_sh.at[d1i], d_sem).wait()

            sgs = (sg0, sg1, sg2)
            for g in range(_CHUNK // 16):
                p16 = pk_v[pl.ds(g * 16, 16)]
                s16 = lax.bitwise_and(p16, 16383)
                d16 = lax.shift_right_logical(p16, 14)
                sgs[g][...] = s16
                dst_v[pl.ds(g * 16, 16)] = d16
                as0 = plsc.load_gather(a_tab, [s16 * 4])
                ad0 = plsc.load_gather(a_tab, [d16 * 4 + 1])
                as1 = plsc.load_gather(a_tab, [s16 * 4 + 2])
                ad1 = plsc.load_gather(a_tab, [d16 * 4 + 3])
                e0 = as0 + ad0
                e1 = as1 + ad1
                e0 = jnp.where(e0 >= 0.0, e0, 0.01 * e0)
                e1 = jnp.where(e1 >= 0.0, e1, 0.01 * e1)
                live = (base + g * 16 + lane) < _E
                ex0b[pl.ds(g * 16, 16)] = jnp.where(live, jnp.exp(e0), 0.0)
                ex1b[pl.ds(g * 16, 16)] = jnp.where(live, jnp.exp(e1), 0.0)
                d0i[pl.ds(g * 16, 16)] = d16 * 2
                d1i[pl.ds(g * 16, 16)] = d16 * 2 + 1
                pltpu.async_copy(z_hbm.at[sgs[g]],
                                 rowbuf.at[pl.ds(g * 16, 16)], sem)
            @pl.when(j < _CHUNKS_PER_TILE - 1)
            def _():
                pltpu.async_copy(
                    pk_hbm.at[pl.ds(base + _CHUNK, _CHUNK)], pk_v, pk_sem)
            pltpu.async_copy(ex0b, den_sh.at[d0i], d_sem, add=True)
            pltpu.async_copy(ex1b, den_sh.at[d1i], d_sem, add=True)
            for g2 in range(_CHUNK // 16):
                pltpu.make_async_copy(z_hbm.at[sgs[g2]],
                                      rowbuf.at[pl.ds(g2 * 16, 16)],
                                      sem).wait()
                exv0 = ex0b[pl.ds(g2 * 16, 16)]
                exv1 = ex1b[pl.ds(g2 * 16, 16)]
                for l in range(16):
                    i = g2 * 16 + l
                    b0 = jnp.broadcast_to(exv0[l], (16,))
                    b1 = jnp.broadcast_to(exv1[l], (16,))
                    for k in range(4):
                        rowbuf[i, pl.ds(k * 16, 16)] = (
                            rowbuf[i, pl.ds(k * 16, 16)] * b0)
                    for k in range(4, 8):
                        rowbuf[i, pl.ds(k * 16, 16)] = (
                            rowbuf[i, pl.ds(k * 16, 16)] * b1)
            pltpu.async_copy(rowbuf, acc_sh.at[dst_v], sc_sem, add=True)
            return 0

        lax.fori_loop(0, _CHUNKS_PER_TILE, chunk, 0)
        pltpu.make_async_copy(rowbuf, acc_sh.at[dst_v], sc_sem).wait()
        pltpu.make_async_copy(ex0b, den_sh.at[d0i], d_sem).wait()
        pltpu.make_async_copy(ex1b, den_sh.at[d1i], d_sem).wait()
        plsc.subcore_barrier()
        pltpu.sync_copy(acc_sh.at[pl.ds(s * _RPT, _RPT)],
                        out_hbm.at[c, pl.ds(s * _RPT, _RPT)])
        pltpu.sync_copy(den_sh.at[pl.ds(s * _DPT, _DPT)],
                        den_hbm.at[c, pl.ds(s * _DPT, _DPT)])
        plsc.subcore_barrier()


def _sc_aggregate(z_f, z_l, a_f, a_l, pk_f, pk_l):
    mesh = plsc.VectorSubcoreMesh(core_axis_name="c", subcore_axis_name="s")
    kern = functools.partial(
        pl.kernel,
        mesh=mesh,
        compiler_params=pltpu.CompilerParams(needs_layout_passes=False),
        out_type=[
            jax.ShapeDtypeStruct((_NC, _NACC, _D), jnp.float32),
            jax.ShapeDtypeStruct((_NC, _NACC, _D), jnp.float32),
            jax.ShapeDtypeStruct((_NC, _DEN), jnp.float32),
            jax.ShapeDtypeStruct((_NC, _DEN), jnp.float32),
        ],
        scratch_types=[
            pltpu.VMEM((_N * 4,), jnp.float32),
            pltpu.VMEM((_CHUNK,), jnp.int32),
            pltpu.VMEM((16,), jnp.int32),
            pltpu.VMEM((16,), jnp.int32),
            pltpu.VMEM((16,), jnp.int32),
            pltpu.VMEM((_CHUNK,), jnp.int32),
            pltpu.VMEM((_CHUNK,), jnp.int32),
            pltpu.VMEM((_CHUNK,), jnp.int32),
            pltpu.VMEM((_CHUNK,), jnp.float32),
            pltpu.VMEM((_CHUNK,), jnp.float32),
            pltpu.VMEM((_CHUNK, _D), jnp.float32),
            pltpu.VMEM((128,), jnp.float32),
            pltpu.SemaphoreType.DMA,
            pltpu.SemaphoreType.DMA,
            pltpu.SemaphoreType.DMA,
            pltpu.SemaphoreType.DMA,
            pltpu.VMEM_SHARED((_NACC, _D), jnp.float32),
            pltpu.VMEM_SHARED((_DEN,), jnp.float32),
        ],
    )(_sc_body)
    return kern(z_f, z_l, a_f, a_l, pk_f, pk_l)


def _combine_body(nf_ref, nl_ref, df_ref, dl_ref, wp1_ref, bp1_ref,
                  wp2_ref, attf_ref, attl_ref, ws_ref):
    i = pl.program_id(0)

    def att_of(a_ref, d_ref):
        n = a_ref[0] + a_ref[1]
        d = d_ref[0] + d_ref[1]
        d0 = d[:, 0:1]
        d1 = d[:, 1:2]
        d0 = jnp.where(d0 > 0.0, d0, 1.0)
        d1 = jnp.where(d1 > 0.0, d1, 1.0)
        return jnp.concatenate([n[:, :64] / d0, n[:, 64:128] / d1], axis=1)

    attf = att_of(nf_ref, df_ref)
    attl = att_of(nl_ref, dl_ref)
    attf_ref[...] = attf
    attl_ref[...] = attl

    wp1 = wp1_ref[...]
    bp1 = bp1_ref[...]
    wp2 = wp2_ref[...]

    def wsum(att):
        t = jnp.tanh(jnp.dot(att, wp1, preferred_element_type=jnp.float32)
                     + bp1)
        return jnp.sum(t * wp2)

    @pl.when(i == 0)
    def _():
        ws_ref[0, 0] = 0.0
        ws_ref[0, 1] = 0.0

    ws_ref[0, 0] = ws_ref[0, 0] + wsum(attf)
    ws_ref[0, 1] = ws_ref[0, 1] + wsum(attl)


def _combine(acc_f, acc_l, den_f, den_l, Wp1T, bp1r, Wp2r):
    blk = 1000
    nblk = _N // blk
    return pl.pallas_call(
        _combine_body,
        grid=(nblk,),
        in_specs=[
            pl.BlockSpec((2, blk, _D), lambda i: (0, i, 0)),
            pl.BlockSpec((2, blk, _D), lambda i: (0, i, 0)),
            pl.BlockSpec((2, blk, 2), lambda i: (0, i, 0)),
            pl.BlockSpec((2, blk, 2), lambda i: (0, i, 0)),
            pl.BlockSpec((_D, _D), lambda i: (0, 0)),
            pl.BlockSpec((1, _D), lambda i: (0, 0)),
            pl.BlockSpec((1, _D), lambda i: (0, 0)),
        ],
        out_specs=[
            pl.BlockSpec((blk, _D), lambda i: (i, 0)),
            pl.BlockSpec((blk, _D), lambda i: (i, 0)),
            pl.BlockSpec((1, 2), lambda i: (0, 0), memory_space=pltpu.SMEM),
        ],
        out_shape=[
            jax.ShapeDtypeStruct((_N, _D), jnp.float32),
            jax.ShapeDtypeStruct((_N, _D), jnp.float32),
            jax.ShapeDtypeStruct((1, 2), jnp.float32),
        ],
    )(acc_f, acc_l, den_f, den_l, Wp1T, bp1r, Wp2r)


def _final_body(attf_ref, attl_ref, beta_ref, out_ref):
    bf = beta_ref[0, 0]
    bl = beta_ref[0, 1]
    out_ref[...] = bf * attf_ref[...] + bl * attl_ref[...]


def _final(attf, attl, beta):
    blk = 1000
    nblk = _N // blk
    return pl.pallas_call(
        _final_body,
        grid=(nblk,),
        in_specs=[
            pl.BlockSpec((blk, _D), lambda i: (i, 0)),
            pl.BlockSpec((blk, _D), lambda i: (i, 0)),
            pl.BlockSpec((1, 2), lambda i: (0, 0), memory_space=pltpu.SMEM),
        ],
        out_specs=pl.BlockSpec((blk, _D), lambda i: (i, 0)),
        out_shape=jax.ShapeDtypeStruct((_N, _D), jnp.float32),
    )(attf, attl, beta)


def _pad_edges(ei):
    pad_n = _EPAD - _E
    ar = jnp.arange(pad_n, dtype=jnp.int32)
    pad_src = ar % _N
    pad_dst = (ar * 37) % _N
    src = jnp.concatenate([ei[0].astype(jnp.int32), pad_src])
    dst = jnp.concatenate([ei[1].astype(jnp.int32), pad_dst])
    return dst * 16384 + src


@jax.jit
def kernel(embed_feat, h_gru, ei_follows, ei_likes, Win_f0, Win_f1, Wa_f0,
           Wa_f1, Win_l0, Win_l1, Wa_l0, Wa_l1, Wp1, bp1, Wp2):
    x = jnp.concatenate([embed_feat, h_gru], axis=-1)
    x_pad = jnp.pad(x, ((0, _NPAD - _N), (0, 0)))

    Wf = jnp.concatenate([Win_f0, Win_f1], axis=0).T
    Wl = jnp.concatenate([Win_l0, Win_l1], axis=0).T
    zvec = jnp.zeros((_OUT,), jnp.float32)

    def amat(Wa0, Wa1):
        c0 = jnp.concatenate([Wa0[0, :_OUT], zvec])
        c1 = jnp.concatenate([Wa0[0, _OUT:], zvec])
        c2 = jnp.concatenate([zvec, Wa1[0, :_OUT]])
        c3 = jnp.concatenate([zvec, Wa1[0, _OUT:]])
        z4 = jnp.zeros((_D, 4), jnp.float32)
        return jnp.concatenate(
            [jnp.stack([c0, c1, c2, c3], axis=1), z4], axis=1)

    Af = amat(Wa_f0, Wa_f1)
    Al = amat(Wa_l0, Wa_l1)

    z_f, z_l, av_f, av_l = _prep(x_pad, Wf, Wl, Af, Al)
    a_f = av_f[:_N, :4].reshape(-1)
    a_l = av_l[:_N, :4].reshape(-1)

    pk_f = _pad_edges(ei_follows)
    pk_l = _pad_edges(ei_likes)

    acc_f, acc_l, den_f, den_l = _sc_aggregate(z_f, z_l, a_f, a_l,
                                               pk_f, pk_l)
    den_f = den_f[:, :2 * _N].reshape(_NC, _N, 2)
    den_l = den_l[:, :2 * _N].reshape(_NC, _N, 2)

    Wp1T = Wp1.T
    bp1r = bp1.reshape(1, _D)
    Wp2r = Wp2.reshape(1, _D)
    attf, attl, ws = _combine(acc_f, acc_l, den_f, den_l, Wp1T, bp1r, Wp2r)

    wf = ws[0, 0] / _N
    wl = ws[0, 1] / _N
    m = jnp.maximum(wf, wl)
    ef = jnp.exp(wf - m)
    el = jnp.exp(wl - m)
    beta = jnp.stack([ef, el]) / (ef + el)
    return _final(attf, attl, beta.reshape(1, 2))


# speedup vs baseline: 30.6023x; 1.0407x over previous
import functools
import jax
import jax.numpy as jnp
from jax import lax
from jax.experimental import pallas as pl
from jax.experimental.pallas import tpu as pltpu
from jax.experimental.pallas import tpu_sc as plsc

_N = 10000
_E = 160000
_D = 128
_OUT = 64
_NPAD = 10240
_NC = 2
_NS = 16
_CHUNK = 48
_CHUNKS_PER_TILE = 105
_EPT = _CHUNK * _CHUNKS_PER_TILE
_EPAD = _NC * _NS * _EPT
_NACC = 10112
_RPT = _NACC // _NS
_DEN = 20480
_DPT = _DEN // _NS


def _prep_body(x_ref, wf_ref, wl_ref, af_ref, al_ref,
               zf_ref, zl_ref, avf_ref, avl_ref):
    x = x_ref[...]
    zf = jnp.dot(x, wf_ref[...], preferred_element_type=jnp.float32)
    zl = jnp.dot(x, wl_ref[...], preferred_element_type=jnp.float32)
    zf_ref[...] = zf
    zl_ref[...] = zl
    avf_ref[...] = jnp.dot(zf, af_ref[...], preferred_element_type=jnp.float32)
    avl_ref[...] = jnp.dot(zl, al_ref[...], preferred_element_type=jnp.float32)


def _prep(x_pad, Wf, Wl, Af, Al):
    nblk = _NPAD // 1024
    return pl.pallas_call(
        _prep_body,
        grid=(nblk,),
        in_specs=[
            pl.BlockSpec((1024, _D), lambda i: (i, 0)),
            pl.BlockSpec((_D, _D), lambda i: (0, 0)),
            pl.BlockSpec((_D, _D), lambda i: (0, 0)),
            pl.BlockSpec((_D, 8), lambda i: (0, 0)),
            pl.BlockSpec((_D, 8), lambda i: (0, 0)),
        ],
        out_specs=[
            pl.BlockSpec((1024, _D), lambda i: (i, 0)),
            pl.BlockSpec((1024, _D), lambda i: (i, 0)),
            pl.BlockSpec((1024, 8), lambda i: (i, 0)),
            pl.BlockSpec((1024, 8), lambda i: (i, 0)),
        ],
        out_shape=[
            jax.ShapeDtypeStruct((_NPAD, _D), jnp.float32),
            jax.ShapeDtypeStruct((_NPAD, _D), jnp.float32),
            jax.ShapeDtypeStruct((_NPAD, 8), jnp.float32),
            jax.ShapeDtypeStruct((_NPAD, 8), jnp.float32),
        ],
    )(x_pad, Wf, Wl, Af, Al)


def _sc_body(zf, zl, af, al, pf, pl2, outf, outl, denf, denl,
             a_tab, pk_v, sg0, sg1, sg2, dst_a, dst_b, d0i, d1i, ex0b, ex1b,
             rowbuf, zbuf, sem, pk_sem, sc_sem, d_sem, acc_sh, den_sh):
    c = lax.axis_index("c")
    s = lax.axis_index("s")
    lane = lax.iota(jnp.int32, 16)
    zero16 = jnp.zeros((16,), jnp.float32)

    for k in range(8):
        zbuf[pl.ds(k * 16, 16)] = zero16

    for r in range(2):
        z_hbm = zf if r == 0 else zl
        a_hbm = af if r == 0 else al
        pk_hbm = pf if r == 0 else pl2
        out_hbm = outf if r == 0 else outl
        den_hbm = denf if r == 0 else denl

        pltpu.sync_copy(a_hbm, a_tab)

        def zrow(i, _):
            for k in range(_D // 16):
                rowbuf[i, pl.ds(k * 16, 16)] = zero16
            return 0
        lax.fori_loop(0, _CHUNK, zrow, 0)
        for t in range(_RPT // _CHUNK):
            pltpu.sync_copy(
                rowbuf, acc_sh.at[pl.ds(s * _RPT + t * _CHUNK, _CHUNK)])
        pltpu.sync_copy(rowbuf.at[pl.ds(0, _RPT % _CHUNK)],
                        acc_sh.at[pl.ds(s * _RPT + _RPT - _RPT % _CHUNK,
                                        _RPT % _CHUNK)])
        for t in range(_DPT // 128):
            pltpu.sync_copy(zbuf, den_sh.at[pl.ds(s * _DPT + t * 128, 128)])
        plsc.subcore_barrier()

        base0 = (c * _NS + s) * _EPT
        pltpu.async_copy(pk_hbm.at[pl.ds(base0, _CHUNK)], pk_v, pk_sem)

        def chunk(j, _):
            base = base0 + j * _CHUNK
            pltpu.make_async_copy(
                pk_hbm.at[pl.ds(base, _CHUNK)], pk_v, pk_sem).wait()

            @pl.when(j > 0)
            def _():
                pltpu.make_async_copy(
                    rowbuf.at[pl.ds(0, 32)], acc_sh.at[dst_a], sc_sem).wait()
                pltpu.make_async_copy(
                    rowbuf.at[pl.ds(32, 16)], acc_sh.at[dst_b], sc_sem).wait()
                pltpu.make_async_copy(ex0b, den_sh.at[d0i], d_sem).wait()
                pltpu.make_async_copy(ex1b, den_sh.at[d1i], d_sem).wait()

            sgs = (sg0, sg1, sg2)
            for g in range(_CHUNK // 16):
                p16 = pk_v[pl.ds(g * 16, 16)]
                s16 = lax.bitwise_and(p16, 16383)
                d16 = lax.shift_right_logical(p16, 14)
                sgs[g][...] = s16
                if g < 2:
                    dst_a[pl.ds(g * 16, 16)] = d16
                else:
                    dst_b[...] = d16
                as0 = plsc.load_gather(a_tab, [s16 * 4])
                ad0 = plsc.load_gather(a_tab, [d16 * 4 + 1])
                as1 = plsc.load_gather(a_tab, [s16 * 4 + 2])
                ad1 = plsc.load_gather(a_tab, [d16 * 4 + 3])
                e0 = as0 + ad0
                e1 = as1 + ad1
                e0 = jnp.where(e0 >= 0.0, e0, 0.01 * e0)
                e1 = jnp.where(e1 >= 0.0, e1, 0.01 * e1)
                live = (base + g * 16 + lane) < _E
                ex0b[pl.ds(g * 16, 16)] = jnp.where(live, jnp.exp(e0), 0.0)
                ex1b[pl.ds(g * 16, 16)] = jnp.where(live, jnp.exp(e1), 0.0)
                d0i[pl.ds(g * 16, 16)] = d16 * 2
                d1i[pl.ds(g * 16, 16)] = d16 * 2 + 1
                pltpu.async_copy(z_hbm.at[sgs[g]],
                                 rowbuf.at[pl.ds(g * 16, 16)], sem)
            @pl.when(j < _CHUNKS_PER_TILE - 1)
            def _():
                pltpu.async_copy(
                    pk_hbm.at[pl.ds(base + _CHUNK, _CHUNK)], pk_v, pk_sem)
            pltpu.async_copy(ex0b, den_sh.at[d0i], d_sem, add=True)
            pltpu.async_copy(ex1b, den_sh.at[d1i], d_sem, add=True)
            for g2 in range(_CHUNK // 16):
                if g2 == 2:
                    pltpu.async_copy(rowbuf.at[pl.ds(0, 32)],
                                     acc_sh.at[dst_a], sc_sem, add=True)
                pltpu.make_async_copy(z_hbm.at[sgs[g2]],
                                      rowbuf.at[pl.ds(g2 * 16, 16)],
                                      sem).wait()
                exv0 = ex0b[pl.ds(g2 * 16, 16)]
                exv1 = ex1b[pl.ds(g2 * 16, 16)]
                for l in range(16):
                    i = g2 * 16 + l
                    b0 = jnp.broadcast_to(exv0[l], (16,))
                    b1 = jnp.broadcast_to(exv1[l], (16,))
                    for k in range(4):
                        rowbuf[i, pl.ds(k * 16, 16)] = (
                            rowbuf[i, pl.ds(k * 16, 16)] * b0)
                    for k in range(4, 8):
                        rowbuf[i, pl.ds(k * 16, 16)] = (
                            rowbuf[i, pl.ds(k * 16, 16)] * b1)
            pltpu.async_copy(rowbuf.at[pl.ds(32, 16)],
                             acc_sh.at[dst_b], sc_sem, add=True)
            return 0

        lax.fori_loop(0, _CHUNKS_PER_TILE, chunk, 0)
        pltpu.make_async_copy(rowbuf.at[pl.ds(0, 32)],
                              acc_sh.at[dst_a], sc_sem).wait()
        pltpu.make_async_copy(rowbuf.at[pl.ds(32, 16)],
                              acc_sh.at[dst_b], sc_sem).wait()
        pltpu.make_async_copy(ex0b, den_sh.at[d0i], d_sem).wait()
        pltpu.make_async_copy(ex1b, den_sh.at[d1i], d_sem).wait()
        plsc.subcore_barrier()
        pltpu.sync_copy(acc_sh.at[pl.ds(s * _RPT, _RPT)],
                        out_hbm.at[c, pl.ds(s * _RPT, _RPT)])
        pltpu.sync_copy(den_sh.at[pl.ds(s * _DPT, _DPT)],
                        den_hbm.at[c, pl.ds(s * _DPT, _DPT)])
        plsc.subcore_barrier()


def _sc_aggregate(z_f, z_l, a_f, a_l, pk_f, pk_l):
    mesh = plsc.VectorSubcoreMesh(core_axis_name="c", subcore_axis_name="s")
    kern = functools.partial(
        pl.kernel,
        mesh=mesh,
        compiler_params=pltpu.CompilerParams(needs_layout_passes=False),
        out_type=[
            jax.ShapeDtypeStruct((_NC, _NACC, _D), jnp.float32),
            jax.ShapeDtypeStruct((_NC, _NACC, _D), jnp.float32),
            jax.ShapeDtypeStruct((_NC, _DEN), jnp.float32),
            jax.ShapeDtypeStruct((_NC, _DEN), jnp.float32),
        ],
        scratch_types=[
            pltpu.VMEM((_N * 4,), jnp.float32),
            pltpu.VMEM((_CHUNK,), jnp.int32),
            pltpu.VMEM((16,), jnp.int32),
            pltpu.VMEM((16,), jnp.int32),
            pltpu.VMEM((16,), jnp.int32),
            pltpu.VMEM((32,), jnp.int32),
            pltpu.VMEM((16,), jnp.int32),
            pltpu.VMEM((_CHUNK,), jnp.int32),
            pltpu.VMEM((_CHUNK,), jnp.int32),
            pltpu.VMEM((_CHUNK,), jnp.float32),
            pltpu.VMEM((_CHUNK,), jnp.float32),
            pltpu.VMEM((_CHUNK, _D), jnp.float32),
            pltpu.VMEM((128,), jnp.float32),
            pltpu.SemaphoreType.DMA,
            pltpu.SemaphoreType.DMA,
            pltpu.SemaphoreType.DMA,
            pltpu.SemaphoreType.DMA,
            pltpu.VMEM_SHARED((_NACC, _D), jnp.float32),
            pltpu.VMEM_SHARED((_DEN,), jnp.float32),
        ],
    )(_sc_body)
    return kern(z_f, z_l, a_f, a_l, pk_f, pk_l)


def _combine_body(nf_ref, nl_ref, df_ref, dl_ref, wp1_ref, bp1_ref,
                  wp2_ref, attf_ref, attl_ref, ws_ref):
    i = pl.program_id(0)

    def att_of(a_ref, d_ref):
        n = a_ref[0] + a_ref[1]
        d = d_ref[0] + d_ref[1]
        d0 = d[:, 0:1]
        d1 = d[:, 1:2]
        d0 = jnp.where(d0 > 0.0, d0, 1.0)
        d1 = jnp.where(d1 > 0.0, d1, 1.0)
        return jnp.concatenate([n[:, :64] / d0, n[:, 64:128] / d1], axis=1)

    attf = att_of(nf_ref, df_ref)
    attl = att_of(nl_ref, dl_ref)
    attf_ref[...] = attf
    attl_ref[...] = attl

    wp1 = wp1_ref[...]
    bp1 = bp1_ref[...]
    wp2 = wp2_ref[...]

    def wsum(att):
        t = jnp.tanh(jnp.dot(att, wp1, preferred_element_type=jnp.float32)
                     + bp1)
        return jnp.sum(t * wp2)

    @pl.when(i == 0)
    def _():
        ws_ref[0, 0] = 0.0
        ws_ref[0, 1] = 0.0

    ws_ref[0, 0] = ws_ref[0, 0] + wsum(attf)
    ws_ref[0, 1] = ws_ref[0, 1] + wsum(attl)


def _combine(acc_f, acc_l, den_f, den_l, Wp1T, bp1r, Wp2r):
    blk = 1000
    nblk = _N // blk
    return pl.pallas_call(
        _combine_body,
        grid=(nblk,),
        in_specs=[
            pl.BlockSpec((2, blk, _D), lambda i: (0, i, 0)),
            pl.BlockSpec((2, blk, _D), lambda i: (0, i, 0)),
            pl.BlockSpec((2, blk, 2), lambda i: (0, i, 0)),
            pl.BlockSpec((2, blk, 2), lambda i: (0, i, 0)),
            pl.BlockSpec((_D, _D), lambda i: (0, 0)),
            pl.BlockSpec((1, _D), lambda i: (0, 0)),
            pl.BlockSpec((1, _D), lambda i: (0, 0)),
        ],
        out_specs=[
            pl.BlockSpec((blk, _D), lambda i: (i, 0)),
            pl.BlockSpec((blk, _D), lambda i: (i, 0)),
            pl.BlockSpec((1, 2), lambda i: (0, 0), memory_space=pltpu.SMEM),
        ],
        out_shape=[
            jax.ShapeDtypeStruct((_N, _D), jnp.float32),
            jax.ShapeDtypeStruct((_N, _D), jnp.float32),
            jax.ShapeDtypeStruct((1, 2), jnp.float32),
        ],
    )(acc_f, acc_l, den_f, den_l, Wp1T, bp1r, Wp2r)


def _final_body(attf_ref, attl_ref, beta_ref, out_ref):
    bf = beta_ref[0, 0]
    bl = beta_ref[0, 1]
    out_ref[...] = bf * attf_ref[...] + bl * attl_ref[...]


def _final(attf, attl, beta):
    blk = 1000
    nblk = _N // blk
    return pl.pallas_call(
        _final_body,
        grid=(nblk,),
        in_specs=[
            pl.BlockSpec((blk, _D), lambda i: (i, 0)),
            pl.BlockSpec((blk, _D), lambda i: (i, 0)),
            pl.BlockSpec((1, 2), lambda i: (0, 0), memory_space=pltpu.SMEM),
        ],
        out_specs=pl.BlockSpec((blk, _D), lambda i: (i, 0)),
        out_shape=jax.ShapeDtypeStruct((_N, _D), jnp.float32),
    )(attf, attl, beta)


def _pad_edges(ei):
    pad_n = _EPAD - _E
    ar = jnp.arange(pad_n, dtype=jnp.int32)
    pad_src = ar % _N
    pad_dst = (ar * 37) % _N
    src = jnp.concatenate([ei[0].astype(jnp.int32), pad_src])
    dst = jnp.concatenate([ei[1].astype(jnp.int32), pad_dst])
    return dst * 16384 + src


@jax.jit
def kernel(embed_feat, h_gru, ei_follows, ei_likes, Win_f0, Win_f1, Wa_f0,
           Wa_f1, Win_l0, Win_l1, Wa_l0, Wa_l1, Wp1, bp1, Wp2):
    x = jnp.concatenate([embed_feat, h_gru], axis=-1)
    x_pad = jnp.pad(x, ((0, _NPAD - _N), (0, 0)))

    Wf = jnp.concatenate([Win_f0, Win_f1], axis=0).T
    Wl = jnp.concatenate([Win_l0, Win_l1], axis=0).T
    zvec = jnp.zeros((_OUT,), jnp.float32)

    def amat(Wa0, Wa1):
        c0 = jnp.concatenate([Wa0[0, :_OUT], zvec])
        c1 = jnp.concatenate([Wa0[0, _OUT:], zvec])
        c2 = jnp.concatenate([zvec, Wa1[0, :_OUT]])
        c3 = jnp.concatenate([zvec, Wa1[0, _OUT:]])
        z4 = jnp.zeros((_D, 4), jnp.float32)
        return jnp.concatenate(
            [jnp.stack([c0, c1, c2, c3], axis=1), z4], axis=1)

    Af = amat(Wa_f0, Wa_f1)
    Al = amat(Wa_l0, Wa_l1)

    z_f, z_l, av_f, av_l = _prep(x_pad, Wf, Wl, Af, Al)
    a_f = av_f[:_N, :4].reshape(-1)
    a_l = av_l[:_N, :4].reshape(-1)

    pk_f = _pad_edges(ei_follows)
    pk_l = _pad_edges(ei_likes)

    acc_f, acc_l, den_f, den_l = _sc_aggregate(z_f, z_l, a_f, a_l,
                                               pk_f, pk_l)
    den_f = den_f[:, :2 * _N].reshape(_NC, _N, 2)
    den_l = den_l[:, :2 * _N].reshape(_NC, _N, 2)

    Wp1T = Wp1.T
    bp1r = bp1.reshape(1, _D)
    Wp2r = Wp2.reshape(1, _D)
    attf, attl, ws = _combine(acc_f, acc_l, den_f, den_l, Wp1T, bp1r, Wp2r)

    wf = ws[0, 0] / _N
    wl = ws[0, 1] / _N
    m = jnp.maximum(wf, wl)
    ef = jnp.exp(wf - m)
    el = jnp.exp(wl - m)
    beta = jnp.stack([ef, el]) / (ef + el)
    return _final(attf, attl, beta.reshape(1, 2))
